# baseline (speedup 1.0000x reference)
import functools

import jax
import jax.numpy as jnp
from jax import lax
from jax.experimental import pallas as pl
from jax.experimental.pallas import tpu as pltpu
from jax.experimental.pallas import tpu_sc as plsc

N = 10000
E = 320000
F_IN = 128
HID = 16
HEADS = 8
NCLS = 32

NW = 32
NB = 320
NPAD = NW * NB
CH = 512
NCHUNK = (E + CH - 1) // CH
EPAD = NCHUNK * CH
DST_SENTINEL = 1 << 29

_HIGH = lax.Precision.HIGHEST


def _dot(a, b):
    return jnp.dot(a, b, precision=_HIGH, preferred_element_type=jnp.float32)


ROWS = 1024
GRID = NPAD // ROWS


def _tc_prologue_body(x_ref, w_ref, a_s_ref, a_d_ref, s16_ref,
                      g_ref, acc0_ref, adst_ref):
    h = _dot(x_ref[...], w_ref[...])
    a_src = _dot(h, a_s_ref[...])
    a_dst = _dot(h, a_d_ref[...])
    e = a_src + a_dst
    w_self = jnp.exp(jnp.where(e >= 0, e, 0.2 * e))
    z8 = jnp.zeros((ROWS, 8), jnp.float32)
    z120 = jnp.zeros((ROWS, 120), jnp.float32)
    g_ref[...] = jnp.concatenate([h, a_src, z120], axis=1)
    acc0_ref[...] = jnp.concatenate(
        [h * _dot(w_self, s16_ref[...]), w_self, z8], axis=1)
    adst_ref[...] = a_dst


def _tc_prologue(xpad, W1, A1s, A1d, S16):
    return pl.pallas_call(
        _tc_prologue_body,
        grid=(GRID,),
        in_specs=[
            pl.BlockSpec((ROWS, F_IN), lambda i: (i, 0)),
            pl.BlockSpec((F_IN, F_IN), lambda i: (0, 0)),
            pl.BlockSpec((F_IN, HEADS), lambda i: (0, 0)),
            pl.BlockSpec((F_IN, HEADS), lambda i: (0, 0)),
            pl.BlockSpec((HEADS, F_IN), lambda i: (0, 0)),
        ],
        out_specs=[
            pl.BlockSpec((ROWS, 256), lambda i: (i, 0)),
            pl.BlockSpec((ROWS, 144), lambda i: (i, 0)),
            pl.BlockSpec((ROWS, 8), lambda i: (i, 0)),
        ],
        out_shape=[
            jax.ShapeDtypeStruct((NPAD, 256), jnp.float32),
            jax.ShapeDtypeStruct((NPAD, 144), jnp.float32),
            jax.ShapeDtypeStruct((NPAD, 8), jnp.float32),
        ],
    )(xpad, W1, A1s, A1d, S16)


def _tc_mid_body(acc_ref, w2_ref, a_s_ref, a_d_ref, s16_ref, s32_ref, b1_ref,
                 g_ref, acc0_ref, adst_ref):
    acc = acc_ref[...]
    num = acc[:, :128]
    den = acc[:, 128:136]
    out1 = num * _dot(1.0 / (den + 1e-16), s16_ref[...]) + b1_ref[...]
    out1 = jnp.where(out1 > 0, out1,
                     jnp.exp(jnp.minimum(out1, 0.0)) - 1.0)
    h2 = _dot(out1, w2_ref[...])
    a_src = _dot(h2, a_s_ref[...])
    a_dst = _dot(h2, a_d_ref[...])
    e = a_src + a_dst
    w_self = jnp.exp(jnp.where(e >= 0, e, 0.2 * e))
    z8 = jnp.zeros((ROWS, 8), jnp.float32)
    z120 = jnp.zeros((ROWS, 120), jnp.float32)
    g_ref[...] = jnp.concatenate([h2, a_src, z120], axis=1)
    acc0_ref[...] = jnp.concatenate(
        [h2 * _dot(w_self, s32_ref[...]), w_self, z8], axis=1)
    adst_ref[...] = a_dst


def _tc_mid(acc1, W2, A2s, A2d, S16, S32, b1):
    return pl.pallas_call(
        _tc_mid_body,
        grid=(GRID,),
        in_specs=[
            pl.BlockSpec((ROWS, 144), lambda i: (i, 0)),
            pl.BlockSpec((128, 256), lambda i: (0, 0)),
            pl.BlockSpec((256, HEADS), lambda i: (0, 0)),
            pl.BlockSpec((256, HEADS), lambda i: (0, 0)),
            pl.BlockSpec((HEADS, 128), lambda i: (0, 0)),
            pl.BlockSpec((HEADS, 256), lambda i: (0, 0)),
            pl.BlockSpec((1, 128), lambda i: (0, 0)),
        ],
        out_specs=[
            pl.BlockSpec((ROWS, 384), lambda i: (i, 0)),
            pl.BlockSpec((ROWS, 272), lambda i: (i, 0)),
            pl.BlockSpec((ROWS, 8), lambda i: (i, 0)),
        ],
        out_shape=[
            jax.ShapeDtypeStruct((NPAD, 384), jnp.float32),
            jax.ShapeDtypeStruct((NPAD, 272), jnp.float32),
            jax.ShapeDtypeStruct((NPAD, 8), jnp.float32),
        ],
    )(acc1, W2, A2s, A2d, S16, S32, b1)


def _tc_epilogue_body(acc_ref, s32_ref, m_ref, b2_ref, out_ref):
    acc = acc_ref[...]
    num = acc[:, :256]
    den = acc[:, 256:264]
    scaled = num * _dot(1.0 / (den + 1e-16), s32_ref[...])
    out_ref[...] = _dot(scaled, m_ref[...]) + b2_ref[...]


def _tc_epilogue(acc2, S32, Mmean, b2):
    return pl.pallas_call(
        _tc_epilogue_body,
        grid=(GRID,),
        in_specs=[
            pl.BlockSpec((ROWS, 272), lambda i: (i, 0)),
            pl.BlockSpec((HEADS, 256), lambda i: (0, 0)),
            pl.BlockSpec((256, NCLS), lambda i: (0, 0)),
            pl.BlockSpec((1, NCLS), lambda i: (0, 0)),
        ],
        out_specs=[pl.BlockSpec((ROWS, NCLS), lambda i: (i, 0))],
        out_shape=[jax.ShapeDtypeStruct((NPAD, NCLS), jnp.float32)],
    )(acc2, S32, Mmean, b2)


def _make_sc_edge(n_feat, nphase):
    W = n_feat + 16
    GW = n_feat + 128
    NBP = NB // nphase
    nch_f = n_feat // 16
    cph = nch_f // HEADS
    mesh = plsc.VectorSubcoreMesh(core_axis_name="c", subcore_axis_name="s")

    @functools.partial(
        pl.kernel,
        out_type=jax.ShapeDtypeStruct((NPAD, W), jnp.float32),
        mesh=mesh,
        compiler_params=pltpu.CompilerParams(needs_layout_passes=False),
        scratch_types=[
            pltpu.VMEM((NBP, W), jnp.float32),
            pltpu.VMEM((NBP * 8 + 16,), jnp.float32),
            pltpu.VMEM((2, 2, CH), jnp.int32),
            pltpu.VMEM((32, GW), jnp.float32),
            pltpu.VMEM((48,), jnp.int32),
            pltpu.VMEM((48,), jnp.int32),
            pltpu.VMEM((32,), jnp.int32),
            pltpu.SemaphoreType.DMA,
            pltpu.SemaphoreType.DMA,
            pltpu.SemaphoreType.DMA,
        ],
    )
    def sc_edge(epack_hbm, g_hbm, acc0_hbm, adst_hbm, out_hbm,
                acc_l, adst_l, ebuf, stag, wl_src, wl_dst,
                gidx, sem_ca, sem_cb, sem_g):
        wid = lax.axis_index("s") * 2 + lax.axis_index("c")
        lane = lax.iota(jnp.int32, 16)
        zi = jnp.zeros((16,), jnp.int32)
        NBAT = CH // 16
        NITER = NCHUNK * NBAT + 2

        for q in range(3):
            wl_src[pl.ds(q * 16, 16)] = zi
            wl_dst[pl.ds(q * 16, 16)] = zi

        def chunk_cp(ci, slot, sem):
            return pltpu.make_async_copy(
                epack_hbm.at[ci], ebuf.at[slot], sem)

        def phase_body(phase, _):
            lo = (phase * NW + wid) * NBP
            pltpu.sync_copy(acc0_hbm.at[pl.ds(lo, NBP)], acc_l)
            pltpu.sync_copy(adst_hbm.at[pl.ds(lo * 8, NBP * 8)],
                            adst_l.at[pl.ds(0, NBP * 8)])
            chunk_cp(0, 0, sem_ca).start()

            def process16(rbase, dvec, base, valid_cnt):
                for ei in range(16):
                    def edge_body(ei=ei):
                        dloc = dvec[ei] - lo
                        adst_v = adst_l[pl.ds(dloc * 8, 16)]
                        asrc_v = stag[rbase + ei, pl.ds(n_feat, 16)]
                        ev = asrc_v + adst_v
                        ev = jnp.where(ev >= 0, ev, 0.2 * ev)
                        w = jnp.exp(ev)
                        wm = jnp.where(lane < HEADS, w, 0.0)
                        plsc.addupdate(acc_l.at[dloc, pl.ds(n_feat, 16)], wm)
                        for c in range(nch_f):
                            bc = jnp.full((16,), w[c // cph], jnp.float32)
                            plsc.addupdate(
                                acc_l.at[dloc, pl.ds(c * 16, 16)],
                                bc * stag[rbase + ei, pl.ds(c * 16, 16)])
                    if valid_cnt is None:
                        edge_body()
                    else:
                        pl.when(jnp.int32(base + ei) < valid_cnt)(edge_body)

            def flush32(valid_cnt):
                gidx[pl.ds(0, 16)] = wl_src[pl.ds(0, 16)]
                gidx[pl.ds(16, 16)] = wl_src[pl.ds(16, 16)]
                cp = pltpu.make_async_copy(g_hbm.at[gidx], stag, sem_g)
                cp.start()
                cp.wait()
                process16(0, wl_dst[pl.ds(0, 16)], 0, valid_cnt)
                process16(16, wl_dst[pl.ds(16, 16)], 16, valid_cnt)
                s16 = wl_src[pl.ds(32, 16)]
                wl_src[pl.ds(0, 16)] = s16
                d16 = wl_dst[pl.ds(32, 16)]
                wl_dst[pl.ds(0, 16)] = d16

            def chunk_body(ci, cnt):
                parity = ci & 1

                def wait_issue(slot):
                    def f():
                        chunk_cp(ci, slot, (sem_ca, sem_cb)[slot]).wait()
                        pl.when(ci + 1 < NCHUNK)(lambda: chunk_cp(
                            ci + 1, 1 - slot, (sem_cb, sem_ca)[slot]).start())
                    return f

                pl.when(parity == 0)(wait_issue(0))
                pl.when(parity == 1)(wait_issue(1))

                def batch_body(b, cnt):
                    sv = ebuf[parity, 0, pl.ds(b * 16, 16)]
                    dv = ebuf[parity, 1, pl.ds(b * 16, 16)]
                    m = (dv >= lo) & (dv < lo + NBP)
                    pc = plsc.cumsum(jnp.where(m, jnp.int32(1), jnp.int32(0)))
                    slotv = cnt + pc - 1
                    plsc.store_scatter(wl_src, [slotv], sv, mask=m)
                    plsc.store_scatter(wl_dst, [slotv], dv, mask=m)
                    cnt = cnt + pc[15]
                    pl.when(cnt >= 32)(lambda: flush32(None))
                    return jnp.where(cnt >= 32, cnt - 32, cnt)

                return lax.fori_loop(0, NBAT, batch_body, cnt)

            cnt = lax.fori_loop(0, NCHUNK, chunk_body, jnp.int32(0))
            pl.when(cnt > 0)(lambda: flush32(cnt))
            pltpu.sync_copy(acc_l, out_hbm.at[pl.ds(lo, NBP)])
            return 0

        lax.fori_loop(0, nphase, phase_body, 0)

    return sc_edge


_sc_edge_l1 = _make_sc_edge(128, 1)
_sc_edge_l2 = _make_sc_edge(256, 2)


def _blockdiag(att):
    a = att[0]
    eye = jnp.eye(HEADS, dtype=jnp.float32)
    return (a[:, :, None] * eye[:, None, :]).reshape(-1, HEADS)


def kernel(x, edge_index, W1, att_src1, att_dst1, b1, W2, att_src2,
           att_dst2, b2):
    ei = edge_index.astype(jnp.int32)
    src = jnp.pad(ei[0], (0, EPAD - E)).reshape(NCHUNK, CH)
    dst = jnp.pad(ei[1], (0, EPAD - E),
                  constant_values=DST_SENTINEL).reshape(NCHUNK, CH)
    epack = jnp.stack([src, dst], axis=1)
    xpad = jnp.pad(x, ((0, NPAD - N), (0, 0)))

    A1s = _blockdiag(att_src1)
    A1d = _blockdiag(att_dst1)
    A2s = _blockdiag(att_src2)
    A2d = _blockdiag(att_dst2)
    S16 = jnp.repeat(jnp.eye(HEADS, dtype=jnp.float32), HID, axis=1)
    S32 = jnp.repeat(jnp.eye(HEADS, dtype=jnp.float32), NCLS, axis=1)
    Mmean = jnp.tile(jnp.eye(NCLS, dtype=jnp.float32), (HEADS, 1)) / HEADS

    g1, acc0_1, adst1 = _tc_prologue(xpad, W1, A1s, A1d, S16)
    acc1 = _sc_edge_l1(epack, g1, acc0_1, adst1.reshape(-1))
    g2, acc0_2, adst2 = _tc_mid(acc1, W2, A2s, A2d, S16, S32,
                                b1.reshape(1, -1))
    acc2 = _sc_edge_l2(epack, g2, acc0_2, adst2.reshape(-1))
    (out,) = _tc_epilogue(acc2, S32, Mmean, b2.reshape(1, -1))
    return out[:N]


# speedup vs baseline: 1.0016x; 1.0016x over previous
import functools

import jax
import jax.numpy as jnp
from jax import lax
from jax.experimental import pallas as pl
from jax.experimental.pallas import tpu as pltpu
from jax.experimental.pallas import tpu_sc as plsc

N = 10000
E = 320000
F_IN = 128
HID = 16
HEADS = 8
NCLS = 32

NW = 32
NB = 320
NPAD = NW * NB
CH = 512
NCHUNK = (E + CH - 1) // CH
EPAD = NCHUNK * CH
DST_SENTINEL = 1 << 29

_HIGH = lax.Precision.HIGHEST


def _dot(a, b):
    return jnp.dot(a, b, precision=_HIGH, preferred_element_type=jnp.float32)


ROWS = 1024
GRID = NPAD // ROWS


def _tc_prologue_body(x_ref, w_ref, a_s_ref, a_d_ref, s16_ref,
                      g_ref, acc0_ref, adst_ref):
    h = _dot(x_ref[...], w_ref[...])
    a_src = _dot(h, a_s_ref[...])
    a_dst = _dot(h, a_d_ref[...])
    e = a_src + a_dst
    w_self = jnp.exp(jnp.where(e >= 0, e, 0.2 * e))
    z8 = jnp.zeros((ROWS, 8), jnp.float32)
    z120 = jnp.zeros((ROWS, 120), jnp.float32)
    g_ref[...] = jnp.concatenate([h, a_src, z120], axis=1)
    acc0_ref[...] = jnp.concatenate(
        [h * _dot(w_self, s16_ref[...]), w_self, z8], axis=1)
    adst_ref[...] = a_dst


def _tc_prologue(xpad, W1, A1s, A1d, S16):
    return pl.pallas_call(
        _tc_prologue_body,
        grid=(GRID,),
        in_specs=[
            pl.BlockSpec((ROWS, F_IN), lambda i: (i, 0)),
            pl.BlockSpec((F_IN, F_IN), lambda i: (0, 0)),
            pl.BlockSpec((F_IN, HEADS), lambda i: (0, 0)),
            pl.BlockSpec((F_IN, HEADS), lambda i: (0, 0)),
            pl.BlockSpec((HEADS, F_IN), lambda i: (0, 0)),
        ],
        out_specs=[
            pl.BlockSpec((ROWS, 256), lambda i: (i, 0)),
            pl.BlockSpec((ROWS, 144), lambda i: (i, 0)),
            pl.BlockSpec((ROWS, 8), lambda i: (i, 0)),
        ],
        out_shape=[
            jax.ShapeDtypeStruct((NPAD, 256), jnp.float32),
            jax.ShapeDtypeStruct((NPAD, 144), jnp.float32),
            jax.ShapeDtypeStruct((NPAD, 8), jnp.float32),
        ],
    )(xpad, W1, A1s, A1d, S16)


def _tc_mid_body(acc_ref, w2_ref, a_s_ref, a_d_ref, s16_ref, s32_ref, b1_ref,
                 g_ref, acc0_ref, adst_ref):
    acc = acc_ref[...]
    num = acc[:, :128]
    den = acc[:, 128:136]
    out1 = num * _dot(1.0 / (den + 1e-16), s16_ref[...]) + b1_ref[...]
    out1 = jnp.where(out1 > 0, out1,
                     jnp.exp(jnp.minimum(out1, 0.0)) - 1.0)
    h2 = _dot(out1, w2_ref[...])
    a_src = _dot(h2, a_s_ref[...])
    a_dst = _dot(h2, a_d_ref[...])
    e = a_src + a_dst
    w_self = jnp.exp(jnp.where(e >= 0, e, 0.2 * e))
    z8 = jnp.zeros((ROWS, 8), jnp.float32)
    z120 = jnp.zeros((ROWS, 120), jnp.float32)
    g_ref[...] = jnp.concatenate([h2, a_src, z120], axis=1)
    acc0_ref[...] = jnp.concatenate(
        [h2 * _dot(w_self, s32_ref[...]), w_self, z8], axis=1)
    adst_ref[...] = a_dst


def _tc_mid(acc1, W2, A2s, A2d, S16, S32, b1):
    return pl.pallas_call(
        _tc_mid_body,
        grid=(GRID,),
        in_specs=[
            pl.BlockSpec((ROWS, 144), lambda i: (i, 0)),
            pl.BlockSpec((128, 256), lambda i: (0, 0)),
            pl.BlockSpec((256, HEADS), lambda i: (0, 0)),
            pl.BlockSpec((256, HEADS), lambda i: (0, 0)),
            pl.BlockSpec((HEADS, 128), lambda i: (0, 0)),
            pl.BlockSpec((HEADS, 256), lambda i: (0, 0)),
            pl.BlockSpec((1, 128), lambda i: (0, 0)),
        ],
        out_specs=[
            pl.BlockSpec((ROWS, 384), lambda i: (i, 0)),
            pl.BlockSpec((ROWS, 272), lambda i: (i, 0)),
            pl.BlockSpec((ROWS, 8), lambda i: (i, 0)),
        ],
        out_shape=[
            jax.ShapeDtypeStruct((NPAD, 384), jnp.float32),
            jax.ShapeDtypeStruct((NPAD, 272), jnp.float32),
            jax.ShapeDtypeStruct((NPAD, 8), jnp.float32),
        ],
    )(acc1, W2, A2s, A2d, S16, S32, b1)


def _tc_epilogue_body(acc_ref, s32_ref, m_ref, b2_ref, out_ref):
    acc = acc_ref[...]
    num = acc[:, :256]
    den = acc[:, 256:264]
    scaled = num * _dot(1.0 / (den + 1e-16), s32_ref[...])
    out_ref[...] = _dot(scaled, m_ref[...]) + b2_ref[...]


def _tc_epilogue(acc2, S32, Mmean, b2):
    return pl.pallas_call(
        _tc_epilogue_body,
        grid=(GRID,),
        in_specs=[
            pl.BlockSpec((ROWS, 272), lambda i: (i, 0)),
            pl.BlockSpec((HEADS, 256), lambda i: (0, 0)),
            pl.BlockSpec((256, NCLS), lambda i: (0, 0)),
            pl.BlockSpec((1, NCLS), lambda i: (0, 0)),
        ],
        out_specs=[pl.BlockSpec((ROWS, NCLS), lambda i: (i, 0))],
        out_shape=[jax.ShapeDtypeStruct((NPAD, NCLS), jnp.float32)],
    )(acc2, S32, Mmean, b2)


def _make_sc_edge(n_feat, nphase):
    W = n_feat + 16
    GW = n_feat + 128
    NBP = NB // nphase
    nch_f = n_feat // 16
    cph = nch_f // HEADS
    mesh = plsc.VectorSubcoreMesh(core_axis_name="c", subcore_axis_name="s")

    @functools.partial(
        pl.kernel,
        out_type=jax.ShapeDtypeStruct((NPAD, W), jnp.float32),
        mesh=mesh,
        compiler_params=pltpu.CompilerParams(needs_layout_passes=False),
        scratch_types=[
            pltpu.VMEM((NBP, W), jnp.float32),
            pltpu.VMEM((NBP * 8 + 16,), jnp.float32),
            pltpu.VMEM((2, 2, CH), jnp.int32),
            pltpu.VMEM((32, GW), jnp.float32),
            pltpu.VMEM((48,), jnp.int32),
            pltpu.VMEM((48,), jnp.int32),
            pltpu.VMEM((32,), jnp.int32),
            pltpu.SemaphoreType.DMA,
            pltpu.SemaphoreType.DMA,
            pltpu.SemaphoreType.DMA,
        ],
    )
    def sc_edge(epack_hbm, g_hbm, acc0_hbm, adst_hbm, out_hbm,
                acc_l, adst_l, ebuf, stag, wl_src, wl_dst,
                gidx, sem_ca, sem_cb, sem_g):
        wid = lax.axis_index("s") * 2 + lax.axis_index("c")
        lane = lax.iota(jnp.int32, 16)
        zi = jnp.zeros((16,), jnp.int32)

        def bcast(vec, j):
            return lax.gather(
                vec, jnp.full((16, 1), j, jnp.int32),
                lax.GatherDimensionNumbers(
                    offset_dims=(), collapsed_slice_dims=(0,),
                    start_index_map=(0,)),
                slice_sizes=(1,),
                mode=lax.GatherScatterMode.PROMISE_IN_BOUNDS)
        NBAT = CH // 16
        NITER = NCHUNK * NBAT + 2

        for q in range(3):
            wl_src[pl.ds(q * 16, 16)] = zi
            wl_dst[pl.ds(q * 16, 16)] = zi

        def chunk_cp(ci, slot, sem):
            return pltpu.make_async_copy(
                epack_hbm.at[ci], ebuf.at[slot], sem)

        def phase_body(phase, _):
            lo = (phase * NW + wid) * NBP
            pltpu.sync_copy(acc0_hbm.at[pl.ds(lo, NBP)], acc_l)
            pltpu.sync_copy(adst_hbm.at[pl.ds(lo * 8, NBP * 8)],
                            adst_l.at[pl.ds(0, NBP * 8)])
            chunk_cp(0, 0, sem_ca).start()

            def process16(rbase, dvec, base, valid_cnt):
                for ei in range(16):
                    def edge_body(ei=ei):
                        dloc = dvec[ei] - lo
                        adst_v = adst_l[pl.ds(dloc * 8, 16)]
                        asrc_v = stag[rbase + ei, pl.ds(n_feat, 16)]
                        ev = asrc_v + adst_v
                        ev = jnp.where(ev >= 0, ev, 0.2 * ev)
                        w = jnp.exp(ev)
                        wm = jnp.where(lane < HEADS, w, 0.0)
                        plsc.addupdate(acc_l.at[dloc, pl.ds(n_feat, 16)], wm)
                        for c in range(nch_f):
                            bc = bcast(w, c // cph)
                            plsc.addupdate(
                                acc_l.at[dloc, pl.ds(c * 16, 16)],
                                bc * stag[rbase + ei, pl.ds(c * 16, 16)])
                    if valid_cnt is None:
                        edge_body()
                    else:
                        pl.when(jnp.int32(base + ei) < valid_cnt)(edge_body)

            def flush32(valid_cnt):
                gidx[pl.ds(0, 16)] = wl_src[pl.ds(0, 16)]
                gidx[pl.ds(16, 16)] = wl_src[pl.ds(16, 16)]
                cp = pltpu.make_async_copy(g_hbm.at[gidx], stag, sem_g)
                cp.start()
                cp.wait()
                process16(0, wl_dst[pl.ds(0, 16)], 0, valid_cnt)
                process16(16, wl_dst[pl.ds(16, 16)], 16, valid_cnt)
                s16 = wl_src[pl.ds(32, 16)]
                wl_src[pl.ds(0, 16)] = s16
                d16 = wl_dst[pl.ds(32, 16)]
                wl_dst[pl.ds(0, 16)] = d16

            def chunk_body(ci, cnt):
                parity = ci & 1

                def wait_issue(slot):
                    def f():
                        chunk_cp(ci, slot, (sem_ca, sem_cb)[slot]).wait()
                        pl.when(ci + 1 < NCHUNK)(lambda: chunk_cp(
                            ci + 1, 1 - slot, (sem_cb, sem_ca)[slot]).start())
                    return f

                pl.when(parity == 0)(wait_issue(0))
                pl.when(parity == 1)(wait_issue(1))

                def batch_body(b, cnt):
                    sv = ebuf[parity, 0, pl.ds(b * 16, 16)]
                    dv = ebuf[parity, 1, pl.ds(b * 16, 16)]
                    m = (dv >= lo) & (dv < lo + NBP)
                    pc = plsc.cumsum(jnp.where(m, jnp.int32(1), jnp.int32(0)))
                    slotv = cnt + pc - 1
                    plsc.store_scatter(wl_src, [slotv], sv, mask=m)
                    plsc.store_scatter(wl_dst, [slotv], dv, mask=m)
                    cnt = cnt + pc[15]
                    pl.when(cnt >= 32)(lambda: flush32(None))
                    return jnp.where(cnt >= 32, cnt - 32, cnt)

                return lax.fori_loop(0, NBAT, batch_body, cnt)

            cnt = lax.fori_loop(0, NCHUNK, chunk_body, jnp.int32(0))
            pl.when(cnt > 0)(lambda: flush32(cnt))
            pltpu.sync_copy(acc_l, out_hbm.at[pl.ds(lo, NBP)])
            return 0

        lax.fori_loop(0, nphase, phase_body, 0)

    return sc_edge


_sc_edge_l1 = _make_sc_edge(128, 1)
_sc_edge_l2 = _make_sc_edge(256, 2)


def _blockdiag(att):
    a = att[0]
    eye = jnp.eye(HEADS, dtype=jnp.float32)
    return (a[:, :, None] * eye[:, None, :]).reshape(-1, HEADS)


def kernel(x, edge_index, W1, att_src1, att_dst1, b1, W2, att_src2,
           att_dst2, b2):
    ei = edge_index.astype(jnp.int32)
    src = jnp.pad(ei[0], (0, EPAD - E)).reshape(NCHUNK, CH)
    dst = jnp.pad(ei[1], (0, EPAD - E),
                  constant_values=DST_SENTINEL).reshape(NCHUNK, CH)
    epack = jnp.stack([src, dst], axis=1)
    xpad = jnp.pad(x, ((0, NPAD - N), (0, 0)))

    A1s = _blockdiag(att_src1)
    A1d = _blockdiag(att_dst1)
    A2s = _blockdiag(att_src2)
    A2d = _blockdiag(att_dst2)
    S16 = jnp.repeat(jnp.eye(HEADS, dtype=jnp.float32), HID, axis=1)
    S32 = jnp.repeat(jnp.eye(HEADS, dtype=jnp.float32), NCLS, axis=1)
    Mmean = jnp.tile(jnp.eye(NCLS, dtype=jnp.float32), (HEADS, 1)) / HEADS

    g1, acc0_1, adst1 = _tc_prologue(xpad, W1, A1s, A1d, S16)
    acc1 = _sc_edge_l1(epack, g1, acc0_1, adst1.reshape(-1))
    g2, acc0_2, adst2 = _tc_mid(acc1, W2, A2s, A2d, S16, S32,
                                b1.reshape(1, -1))
    acc2 = _sc_edge_l2(epack, g2, acc0_2, adst2.reshape(-1))
    (out,) = _tc_epilogue(acc2, S32, Mmean, b2.reshape(1, -1))
    return out[:N]


# speedup vs baseline: 1.0129x; 1.0113x over previous
import functools

import jax
import jax.numpy as jnp
from jax import lax
from jax.experimental import pallas as pl
from jax.experimental.pallas import tpu as pltpu
from jax.experimental.pallas import tpu_sc as plsc

N = 10000
E = 320000
F_IN = 128
HID = 16
HEADS = 8
NCLS = 32

NW = 32
NB = 320
NPAD = NW * NB
CH = 512
NCHUNK = (E + CH - 1) // CH
EPAD = NCHUNK * CH
DST_SENTINEL = 1 << 29

_HIGH = lax.Precision.HIGHEST


def _dot(a, b):
    return jnp.dot(a, b, precision=_HIGH, preferred_element_type=jnp.float32)


ROWS = 1024
GRID = NPAD // ROWS


def _tc_prologue_body(x_ref, w_ref, a_s_ref, a_d_ref, s16_ref,
                      g_ref, acc0_ref, adst_ref):
    h = _dot(x_ref[...], w_ref[...])
    a_src = _dot(h, a_s_ref[...])
    a_dst = _dot(h, a_d_ref[...])
    e = a_src + a_dst
    w_self = jnp.exp(jnp.where(e >= 0, e, 0.2 * e))
    z8 = jnp.zeros((ROWS, 8), jnp.float32)
    z120 = jnp.zeros((ROWS, 120), jnp.float32)
    g_ref[...] = jnp.concatenate([h, a_src, z120], axis=1)
    acc0_ref[...] = jnp.concatenate(
        [h * _dot(w_self, s16_ref[...]), w_self, z8], axis=1)
    adst_ref[...] = a_dst


def _tc_prologue(xpad, W1, A1s, A1d, S16):
    return pl.pallas_call(
        _tc_prologue_body,
        grid=(GRID,),
        in_specs=[
            pl.BlockSpec((ROWS, F_IN), lambda i: (i, 0)),
            pl.BlockSpec((F_IN, F_IN), lambda i: (0, 0)),
            pl.BlockSpec((F_IN, HEADS), lambda i: (0, 0)),
            pl.BlockSpec((F_IN, HEADS), lambda i: (0, 0)),
            pl.BlockSpec((HEADS, F_IN), lambda i: (0, 0)),
        ],
        out_specs=[
            pl.BlockSpec((ROWS, 256), lambda i: (i, 0)),
            pl.BlockSpec((ROWS, 144), lambda i: (i, 0)),
            pl.BlockSpec((ROWS, 8), lambda i: (i, 0)),
        ],
        out_shape=[
            jax.ShapeDtypeStruct((NPAD, 256), jnp.float32),
            jax.ShapeDtypeStruct((NPAD, 144), jnp.float32),
            jax.ShapeDtypeStruct((NPAD, 8), jnp.float32),
        ],
    )(xpad, W1, A1s, A1d, S16)


def _tc_mid_body(acc_ref, w2_ref, a_s_ref, a_d_ref, s16_ref, s32_ref, b1_ref,
                 g_ref, acc0_ref, adst_ref):
    acc = acc_ref[...]
    num = acc[:, :128]
    den = acc[:, 128:136]
    out1 = num * _dot(1.0 / (den + 1e-16), s16_ref[...]) + b1_ref[...]
    out1 = jnp.where(out1 > 0, out1,
                     jnp.exp(jnp.minimum(out1, 0.0)) - 1.0)
    h2 = _dot(out1, w2_ref[...])
    a_src = _dot(h2, a_s_ref[...])
    a_dst = _dot(h2, a_d_ref[...])
    e = a_src + a_dst
    w_self = jnp.exp(jnp.where(e >= 0, e, 0.2 * e))
    z8 = jnp.zeros((ROWS, 8), jnp.float32)
    z120 = jnp.zeros((ROWS, 120), jnp.float32)
    g_ref[...] = jnp.concatenate([h2, a_src, z120], axis=1)
    acc0_ref[...] = jnp.concatenate(
        [h2 * _dot(w_self, s32_ref[...]), w_self, z8], axis=1)
    adst_ref[...] = a_dst


def _tc_mid(acc1, W2, A2s, A2d, S16, S32, b1):
    return pl.pallas_call(
        _tc_mid_body,
        grid=(GRID,),
        in_specs=[
            pl.BlockSpec((ROWS, 144), lambda i: (i, 0)),
            pl.BlockSpec((128, 256), lambda i: (0, 0)),
            pl.BlockSpec((256, HEADS), lambda i: (0, 0)),
            pl.BlockSpec((256, HEADS), lambda i: (0, 0)),
            pl.BlockSpec((HEADS, 128), lambda i: (0, 0)),
            pl.BlockSpec((HEADS, 256), lambda i: (0, 0)),
            pl.BlockSpec((1, 128), lambda i: (0, 0)),
        ],
        out_specs=[
            pl.BlockSpec((ROWS, 384), lambda i: (i, 0)),
            pl.BlockSpec((ROWS, 272), lambda i: (i, 0)),
            pl.BlockSpec((ROWS, 8), lambda i: (i, 0)),
        ],
        out_shape=[
            jax.ShapeDtypeStruct((NPAD, 384), jnp.float32),
            jax.ShapeDtypeStruct((NPAD, 272), jnp.float32),
            jax.ShapeDtypeStruct((NPAD, 8), jnp.float32),
        ],
    )(acc1, W2, A2s, A2d, S16, S32, b1)


def _tc_epilogue_body(acc_ref, s32_ref, m_ref, b2_ref, out_ref):
    acc = acc_ref[...]
    num = acc[:, :256]
    den = acc[:, 256:264]
    scaled = num * _dot(1.0 / (den + 1e-16), s32_ref[...])
    out_ref[...] = _dot(scaled, m_ref[...]) + b2_ref[...]


def _tc_epilogue(acc2, S32, Mmean, b2):
    return pl.pallas_call(
        _tc_epilogue_body,
        grid=(GRID,),
        in_specs=[
            pl.BlockSpec((ROWS, 272), lambda i: (i, 0)),
            pl.BlockSpec((HEADS, 256), lambda i: (0, 0)),
            pl.BlockSpec((256, NCLS), lambda i: (0, 0)),
            pl.BlockSpec((1, NCLS), lambda i: (0, 0)),
        ],
        out_specs=[pl.BlockSpec((ROWS, NCLS), lambda i: (i, 0))],
        out_shape=[jax.ShapeDtypeStruct((NPAD, NCLS), jnp.float32)],
    )(acc2, S32, Mmean, b2)


def _make_sc_edge(n_feat, nphase):
    W = n_feat + 16
    GW = n_feat + 128
    NBP = NB // nphase
    nch_f = n_feat // 16
    cph = nch_f // HEADS
    mesh = plsc.VectorSubcoreMesh(core_axis_name="c", subcore_axis_name="s")

    @functools.partial(
        pl.kernel,
        out_type=jax.ShapeDtypeStruct((NPAD, W), jnp.float32),
        mesh=mesh,
        compiler_params=pltpu.CompilerParams(needs_layout_passes=False),
        scratch_types=[
            pltpu.VMEM((NBP, W), jnp.float32),
            pltpu.VMEM((NBP * 8 + 16,), jnp.float32),
            pltpu.VMEM((2, 2, CH), jnp.int32),
            pltpu.VMEM((16, GW), jnp.float32),
            pltpu.VMEM((16, GW), jnp.float32),
            pltpu.VMEM((48,), jnp.int32),
            pltpu.VMEM((48,), jnp.int32),
            pltpu.VMEM((16,), jnp.int32),
            pltpu.VMEM((16,), jnp.int32),
            pltpu.SemaphoreType.DMA,
            pltpu.SemaphoreType.DMA,
            pltpu.SemaphoreType.DMA,
            pltpu.SemaphoreType.DMA,
        ],
    )
    def sc_edge(epack_hbm, g_hbm, acc0_hbm, adst_hbm, out_hbm,
                acc_l, adst_l, ebuf, stag_a, stag_b, wl_src, wl_dst,
                gidx_a, gidx_b, sem_ca, sem_cb, sem_ga, sem_gb):
        wid = lax.axis_index("s") * 2 + lax.axis_index("c")
        lane = lax.iota(jnp.int32, 16)
        zi = jnp.zeros((16,), jnp.int32)
        NBAT = CH // 16
        NITER = NCHUNK * NBAT + 2

        for q in range(3):
            wl_src[pl.ds(q * 16, 16)] = zi
            wl_dst[pl.ds(q * 16, 16)] = zi

        def chunk_cp(ci, slot, sem):
            return pltpu.make_async_copy(
                epack_hbm.at[ci], ebuf.at[slot], sem)

        def phase_body(phase, _):
            lo = (phase * NW + wid) * NBP
            pltpu.sync_copy(acc0_hbm.at[pl.ds(lo, NBP)], acc_l)
            pltpu.sync_copy(adst_hbm.at[pl.ds(lo * 8, NBP * 8)],
                            adst_l.at[pl.ds(0, NBP * 8)])
            chunk_cp(0, 0, sem_ca).start()

            def process16(stg, dvec, base, valid_cnt):
                for ei in range(16):
                    def edge_body(ei=ei):
                        dloc = dvec[ei] - lo
                        adst_v = adst_l[pl.ds(dloc * 8, 16)]
                        asrc_v = stg[ei, pl.ds(n_feat, 16)]
                        ev = asrc_v + adst_v
                        ev = jnp.where(ev >= 0, ev, 0.2 * ev)
                        w = jnp.exp(ev)
                        wm = jnp.where(lane < HEADS, w, 0.0)
                        plsc.addupdate(acc_l.at[dloc, pl.ds(n_feat, 16)], wm)
                        for c in range(nch_f):
                            bc = jnp.full((16,), w[c // cph], jnp.float32)
                            plsc.addupdate(
                                acc_l.at[dloc, pl.ds(c * 16, 16)],
                                bc * stg[ei, pl.ds(c * 16, 16)])
                    if valid_cnt is None:
                        edge_body()
                    else:
                        pl.when(jnp.int32(base + ei) < valid_cnt)(edge_body)

            def flush32(valid_cnt):
                gidx_a[pl.ds(0, 16)] = wl_src[pl.ds(0, 16)]
                cp_a = pltpu.make_async_copy(g_hbm.at[gidx_a], stag_a, sem_ga)
                cp_a.start()
                gidx_b[pl.ds(0, 16)] = wl_src[pl.ds(16, 16)]
                cp_b = pltpu.make_async_copy(g_hbm.at[gidx_b], stag_b, sem_gb)
                cp_b.start()
                cp_a.wait()
                process16(stag_a, wl_dst[pl.ds(0, 16)], 0, valid_cnt)
                cp_b.wait()
                process16(stag_b, wl_dst[pl.ds(16, 16)], 16, valid_cnt)
                s16 = wl_src[pl.ds(32, 16)]
                wl_src[pl.ds(0, 16)] = s16
                d16 = wl_dst[pl.ds(32, 16)]
                wl_dst[pl.ds(0, 16)] = d16

            def chunk_body(ci, cnt):
                parity = ci & 1

                def wait_issue(slot):
                    def f():
                        chunk_cp(ci, slot, (sem_ca, sem_cb)[slot]).wait()
                        pl.when(ci + 1 < NCHUNK)(lambda: chunk_cp(
                            ci + 1, 1 - slot, (sem_cb, sem_ca)[slot]).start())
                    return f

                pl.when(parity == 0)(wait_issue(0))
                pl.when(parity == 1)(wait_issue(1))

                def batch_body(b, cnt):
                    sv = ebuf[parity, 0, pl.ds(b * 16, 16)]
                    dv = ebuf[parity, 1, pl.ds(b * 16, 16)]
                    m = (dv >= lo) & (dv < lo + NBP)
                    pc = plsc.cumsum(jnp.where(m, jnp.int32(1), jnp.int32(0)))
                    slotv = cnt + pc - 1
                    plsc.store_scatter(wl_src, [slotv], sv, mask=m)
                    plsc.store_scatter(wl_dst, [slotv], dv, mask=m)
                    cnt = cnt + pc[15]
                    pl.when(cnt >= 32)(lambda: flush32(None))
                    return jnp.where(cnt >= 32, cnt - 32, cnt)

                return lax.fori_loop(0, NBAT, batch_body, cnt)

            cnt = lax.fori_loop(0, NCHUNK, chunk_body, jnp.int32(0))
            pl.when(cnt > 0)(lambda: flush32(cnt))
            pltpu.sync_copy(acc_l, out_hbm.at[pl.ds(lo, NBP)])
            return 0

        lax.fori_loop(0, nphase, phase_body, 0)

    return sc_edge


_sc_edge_l1 = _make_sc_edge(128, 1)
_sc_edge_l2 = _make_sc_edge(256, 2)


def _blockdiag(att):
    a = att[0]
    eye = jnp.eye(HEADS, dtype=jnp.float32)
    return (a[:, :, None] * eye[:, None, :]).reshape(-1, HEADS)


def kernel(x, edge_index, W1, att_src1, att_dst1, b1, W2, att_src2,
           att_dst2, b2):
    ei = edge_index.astype(jnp.int32)
    src = jnp.pad(ei[0], (0, EPAD - E)).reshape(NCHUNK, CH)
    dst = jnp.pad(ei[1], (0, EPAD - E),
                  constant_values=DST_SENTINEL).reshape(NCHUNK, CH)
    epack = jnp.stack([src, dst], axis=1)
    xpad = jnp.pad(x, ((0, NPAD - N), (0, 0)))

    A1s = _blockdiag(att_src1)
    A1d = _blockdiag(att_dst1)
    A2s = _blockdiag(att_src2)
    A2d = _blockdiag(att_dst2)
    S16 = jnp.repeat(jnp.eye(HEADS, dtype=jnp.float32), HID, axis=1)
    S32 = jnp.repeat(jnp.eye(HEADS, dtype=jnp.float32), NCLS, axis=1)
    Mmean = jnp.tile(jnp.eye(NCLS, dtype=jnp.float32), (HEADS, 1)) / HEADS

    g1, acc0_1, adst1 = _tc_prologue(xpad, W1, A1s, A1d, S16)
    acc1 = _sc_edge_l1(epack, g1, acc0_1, adst1.reshape(-1))
    g2, acc0_2, adst2 = _tc_mid(acc1, W2, A2s, A2d, S16, S32,
                                b1.reshape(1, -1))
    acc2 = _sc_edge_l2(epack, g2, acc0_2, adst2.reshape(-1))
    (out,) = _tc_epilogue(acc2, S32, Mmean, b2.reshape(1, -1))
    return out[:N]
